# baseline (speedup 1.0000x reference)
import math

import jax
import jax.numpy as jnp
from jax import lax
from jax.experimental import pallas as pl
from jax.experimental.pallas import tpu as pltpu
from jax.experimental.pallas import tpu_sc as plsc

NC = 2
NS = 16
NW = NC * NS
B = 128
NBUF = 2
CHUNK = 40


def _mm_body(x_ref, w_ref, o_ref):
    o_ref[...] = lax.dot_general(
        x_ref[...], w_ref[...], (((1,), (0,)), ((), ())),
        precision=lax.Precision.HIGHEST, preferred_element_type=jnp.float32)


def _matmul(x, W, row_block):
    N, D = x.shape
    return pl.pallas_call(
        _mm_body,
        grid=(N // row_block,),
        in_specs=[pl.BlockSpec((row_block, D), lambda i: (i, 0)),
                  pl.BlockSpec((D, D), lambda i: (0, 0))],
        out_specs=pl.BlockSpec((row_block, D), lambda i: (i, 0)),
        out_shape=jax.ShapeDtypeStruct((N, D), jnp.float32),
    )(x, W)


def _mid_body(sp_ref, g_ref, dgp_ref, x_ref, b_ref, w_ref, o_ref):
    s = sp_ref[0] + sp_ref[1] - g_ref[...]
    deg = (dgp_ref[0, :, 0:1] + dgp_ref[1, :, 0:1]
           - 2.0 * x_ref[:, 0:1] + 1.0)
    h = jnp.maximum(s / deg + b_ref[...], 0.0)
    o_ref[...] = lax.dot_general(
        h, w_ref[...], (((1,), (0,)), ((), ())),
        precision=lax.Precision.HIGHEST, preferred_element_type=jnp.float32)


def _tc_mid(sp, g, dgp, x, b2d, W, row_block):
    N, D = g.shape
    return pl.pallas_call(
        _mid_body,
        grid=(N // row_block,),
        in_specs=[pl.BlockSpec((2, row_block, D), lambda i: (0, i, 0)),
                  pl.BlockSpec((row_block, D), lambda i: (i, 0)),
                  pl.BlockSpec((2, row_block, D), lambda i: (0, i, 0)),
                  pl.BlockSpec((row_block, D), lambda i: (i, 0)),
                  pl.BlockSpec((1, D), lambda i: (0, 0)),
                  pl.BlockSpec((D, D), lambda i: (0, 0))],
        out_specs=pl.BlockSpec((row_block, D), lambda i: (i, 0)),
        out_shape=jax.ShapeDtypeStruct((N, D), jnp.float32),
    )(sp, g, dgp, x, b2d, W)


def _fin_body(sp_ref, g_ref, dgp_ref, x_ref, b_ref, o_ref):
    s = sp_ref[0] + sp_ref[1] - g_ref[...]
    deg = (dgp_ref[0, :, 0:1] + dgp_ref[1, :, 0:1]
           - 2.0 * x_ref[:, 0:1] + 1.0)
    o_ref[...] = s / deg + b_ref[...]


def _tc_final(sp, g, dgp, x, b2d, row_block):
    N, D = g.shape
    return pl.pallas_call(
        _fin_body,
        grid=(N // row_block,),
        in_specs=[pl.BlockSpec((2, row_block, D), lambda i: (0, i, 0)),
                  pl.BlockSpec((row_block, D), lambda i: (i, 0)),
                  pl.BlockSpec((2, row_block, D), lambda i: (0, i, 0)),
                  pl.BlockSpec((row_block, D), lambda i: (i, 0)),
                  pl.BlockSpec((1, D), lambda i: (0, 0))],
        out_specs=pl.BlockSpec((row_block, D), lambda i: (i, 0)),
        out_shape=jax.ShapeDtypeStruct((N, D), jnp.float32),
    )(sp, g, dgp, x, b2d)


def _sc_messages(g, pkJ):
    N, D = g.shape
    nb = pkJ.shape[1]
    rows_per = N // NS
    mesh = plsc.VectorSubcoreMesh(core_axis_name="c", subcore_axis_name="s")

    scratch = [pltpu.VMEM((nb, B), jnp.int32),
               pltpu.VMEM((4, B), jnp.int32),
               pltpu.VMEM((B, D), jnp.float32),
               pltpu.VMEM((B, D), jnp.float32),
               pltpu.SemaphoreType.DMA,
               pltpu.SemaphoreType.DMA,
               pltpu.VMEM_SHARED((N, D), jnp.float32)]

    def body(g_h, pkJ_h, out_h, pk_v, idx_v, rows0, rows1, sem0, sem1,
             acc_sh):
        c = lax.axis_index("c")
        s = lax.axis_index("s")
        w = c * NS + s
        pltpu.sync_copy(pkJ_h.at[w], pk_v)
        r0 = s * rows_per
        pltpu.sync_copy(g_h.at[pl.ds(r0, rows_per)],
                        acc_sh.at[pl.ds(r0, rows_per)])
        plsc.subcore_barrier()

        def unpack(j, lane):
            for k in range(B // 16):
                v = pk_v[j, pl.ds(k * 16, 16)]
                idx_v[2 * lane, pl.ds(k * 16, 16)] = jnp.bitwise_and(
                    v, 0xFFFF)
                idx_v[2 * lane + 1, pl.ds(k * 16, 16)] = jnp.right_shift(
                    v, 16)

        def fire(j, lane, rows, sem):
            unpack(j, lane)
            pltpu.async_copy(g_h.at[idx_v.at[2 * lane]], rows, sem)

        def drain_scat(lane, rows, sem):
            pltpu.make_async_copy(g_h.at[pl.ds(0, B)], rows, sem).wait()
            pltpu.sync_copy(rows, acc_sh.at[idx_v.at[2 * lane + 1]],
                            add=True)

        fire(0, 0, rows0, sem0)
        fire(1, 1, rows1, sem1)

        def step(p, carry):
            j = 2 * p
            drain_scat(0, rows0, sem0)
            fire(j + 2, 0, rows0, sem0)
            drain_scat(1, rows1, sem1)
            fire(j + 3, 1, rows1, sem1)
            return carry

        lax.fori_loop(0, nb // 2 - 1, step, 0)
        drain_scat(0, rows0, sem0)
        drain_scat(1, rows1, sem1)
        plsc.subcore_barrier()
        pltpu.sync_copy(acc_sh.at[pl.ds(r0, rows_per)],
                        out_h.at[c, pl.ds(r0, rows_per)])

    fn = pl.kernel(body,
                   out_type=jax.ShapeDtypeStruct((2, N, D), jnp.float32),
                   mesh=mesh, scratch_types=scratch)
    return fn(g, pkJ)


def _sc_degrees(x, dstJ, ones_hb):
    N, D = x.shape
    nb = dstJ.shape[1]
    rows_per = N // NS
    mesh = plsc.VectorSubcoreMesh(core_axis_name="c", subcore_axis_name="s")

    scratch = [pltpu.VMEM((nb, B), jnp.int32),
               pltpu.VMEM((B, D), jnp.float32),
               pltpu.VMEM_SHARED((N, D), jnp.float32)]

    def body(x_h, dstJ_h, ones_h, out_h, dst_v, ones_v, acc_sh):
        c = lax.axis_index("c")
        s = lax.axis_index("s")
        w = c * NS + s
        pltpu.sync_copy(dstJ_h.at[w], dst_v)
        pltpu.sync_copy(ones_h, ones_v)
        r0 = s * rows_per
        pltpu.sync_copy(x_h.at[pl.ds(r0, rows_per)],
                        acc_sh.at[pl.ds(r0, rows_per)])
        plsc.subcore_barrier()

        def step(j, carry):
            pltpu.sync_copy(ones_v, acc_sh.at[dst_v.at[j]], add=True)
            return carry

        lax.fori_loop(0, nb, step, 0)
        plsc.subcore_barrier()
        pltpu.sync_copy(acc_sh.at[pl.ds(r0, rows_per)],
                        out_h.at[c, pl.ds(r0, rows_per)])

    fn = pl.kernel(body,
                   out_type=jax.ShapeDtypeStruct((2, N, D), jnp.float32),
                   mesh=mesh, scratch_types=scratch)
    return fn(x, dstJ, ones_hb)


def kernel(x, edge_index, W1, b1, W2, b2):
    N, D = x.shape
    E = edge_index.shape[1]

    chunk = -(-(-(-N // NS)) // 8) * 8
    Np = chunk * NS
    if Np != N:
        x = jnp.concatenate([x, jnp.zeros((Np - N, D), x.dtype)])

    nb = max(4, -(-math.ceil(E / (NW * B)) // 2) * 2)
    ep = NW * B * nb
    src = edge_index[0]
    dst = edge_index[1]
    if ep != E:
        src = jnp.concatenate([src, jnp.zeros((ep - E,), jnp.int32)])
        dst = jnp.concatenate([dst, jnp.full((ep - E,), N, jnp.int32)])
    dstJ = dst.reshape(NW, nb, B)
    pkJ = jnp.bitwise_or(src, jnp.left_shift(dst, 16)).reshape(NW, nb, B)

    b1_2d = b1.reshape(1, D)
    b2_2d = b2.reshape(1, D)
    rb = Np // 4 if Np % 4 == 0 else Np

    ones_hb = jnp.ones((B, D), jnp.float32)
    dgp = _sc_degrees(x, dstJ, ones_hb)
    g1 = _matmul(x, W1, rb)
    s1p = _sc_messages(g1, pkJ)
    g2 = _tc_mid(s1p, g1, dgp, x, b1_2d, W2, rb)
    s2p = _sc_messages(g2, pkJ)
    return _tc_final(s2p, g2, dgp, x, b2_2d, rb)[:N]


# speedup vs baseline: 1.2122x; 1.2122x over previous
import math

import jax
import jax.numpy as jnp
from jax import lax
from jax.experimental import pallas as pl
from jax.experimental.pallas import tpu as pltpu
from jax.experimental.pallas import tpu_sc as plsc

NC = 2
NS = 16
NW = NC * NS
B = 128
NBUF = 2
CHUNK = 40


def _mm_body(x_ref, w_ref, o_ref):
    o_ref[...] = lax.dot_general(
        x_ref[...], w_ref[...], (((1,), (0,)), ((), ())),
        precision=lax.Precision.HIGHEST, preferred_element_type=jnp.float32)


def _matmul(x, W, row_block):
    N, D = x.shape
    return pl.pallas_call(
        _mm_body,
        grid=(N // row_block,),
        in_specs=[pl.BlockSpec((row_block, D), lambda i: (i, 0)),
                  pl.BlockSpec((D, D), lambda i: (0, 0))],
        out_specs=pl.BlockSpec((row_block, D), lambda i: (i, 0)),
        out_shape=jax.ShapeDtypeStruct((N, D), jnp.float32),
    )(x, W)


def _mid_body(sp_ref, g_ref, dgp_ref, x_ref, b_ref, w_ref, o_ref):
    s = sp_ref[0] + sp_ref[1] - g_ref[...]
    deg = (dgp_ref[0, :, 0:1] + dgp_ref[1, :, 0:1]
           - 2.0 * x_ref[:, 0:1] + 1.0)
    h = jnp.maximum(s / deg + b_ref[...], 0.0)
    o_ref[...] = lax.dot_general(
        h, w_ref[...], (((1,), (0,)), ((), ())),
        precision=lax.Precision.HIGHEST, preferred_element_type=jnp.float32)


def _tc_mid(sp, g, dgp, x, b2d, W, row_block):
    N, D = g.shape
    return pl.pallas_call(
        _mid_body,
        grid=(N // row_block,),
        in_specs=[pl.BlockSpec((2, row_block, D), lambda i: (0, i, 0)),
                  pl.BlockSpec((row_block, D), lambda i: (i, 0)),
                  pl.BlockSpec((2, row_block, D), lambda i: (0, i, 0)),
                  pl.BlockSpec((row_block, D), lambda i: (i, 0)),
                  pl.BlockSpec((1, D), lambda i: (0, 0)),
                  pl.BlockSpec((D, D), lambda i: (0, 0))],
        out_specs=pl.BlockSpec((row_block, D), lambda i: (i, 0)),
        out_shape=jax.ShapeDtypeStruct((N, D), jnp.float32),
    )(sp, g, dgp, x, b2d, W)


def _fin_body(sp_ref, g_ref, dgp_ref, x_ref, b_ref, o_ref):
    s = sp_ref[0] + sp_ref[1] - g_ref[...]
    deg = (dgp_ref[0, :, 0:1] + dgp_ref[1, :, 0:1]
           - 2.0 * x_ref[:, 0:1] + 1.0)
    o_ref[...] = s / deg + b_ref[...]


def _tc_final(sp, g, dgp, x, b2d, row_block):
    N, D = g.shape
    return pl.pallas_call(
        _fin_body,
        grid=(N // row_block,),
        in_specs=[pl.BlockSpec((2, row_block, D), lambda i: (0, i, 0)),
                  pl.BlockSpec((row_block, D), lambda i: (i, 0)),
                  pl.BlockSpec((2, row_block, D), lambda i: (0, i, 0)),
                  pl.BlockSpec((row_block, D), lambda i: (i, 0)),
                  pl.BlockSpec((1, D), lambda i: (0, 0))],
        out_specs=pl.BlockSpec((row_block, D), lambda i: (i, 0)),
        out_shape=jax.ShapeDtypeStruct((N, D), jnp.float32),
    )(sp, g, dgp, x, b2d)


def _sc_messages(g, srcJ, dstJ):
    N, D = g.shape
    nb = srcJ.shape[1]
    rows_per = N // NS
    mesh = plsc.VectorSubcoreMesh(core_axis_name="c", subcore_axis_name="s")

    scratch = [pltpu.VMEM((nb, B), jnp.int32),
               pltpu.VMEM((nb, B), jnp.int32),
               pltpu.VMEM((B, D), jnp.float32),
               pltpu.SemaphoreType.DMA,
               pltpu.SemaphoreType.DMA,
               pltpu.VMEM_SHARED((N, D), jnp.float32)]

    def body(g_h, srcJ_h, dstJ_h, out_h, src_v, dst_v, rows_v, sem0, sem1,
             acc_sh):
        c = lax.axis_index("c")
        s = lax.axis_index("s")
        w = c * NS + s
        pltpu.sync_copy(srcJ_h.at[w], src_v)
        pltpu.sync_copy(dstJ_h.at[w], dst_v)
        r0 = s * rows_per
        pltpu.sync_copy(g_h.at[pl.ds(r0, rows_per)],
                        acc_sh.at[pl.ds(r0, rows_per)])
        plsc.subcore_barrier()

        H = B // 2

        def step(j, carry):
            h0 = pltpu.async_copy(g_h.at[src_v.at[j, pl.ds(0, H)]],
                                  rows_v.at[pl.ds(0, H)], sem0)
            h1 = pltpu.async_copy(g_h.at[src_v.at[j, pl.ds(H, H)]],
                                  rows_v.at[pl.ds(H, H)], sem1)
            h0.wait()
            h1.wait()
            pltpu.sync_copy(rows_v, acc_sh.at[dst_v.at[j]], add=True)
            return carry

        lax.fori_loop(0, nb, step, 0)
        plsc.subcore_barrier()
        pltpu.sync_copy(acc_sh.at[pl.ds(r0, rows_per)],
                        out_h.at[c, pl.ds(r0, rows_per)])

    fn = pl.kernel(body,
                   out_type=jax.ShapeDtypeStruct((2, N, D), jnp.float32),
                   mesh=mesh, scratch_types=scratch)
    return fn(g, srcJ, dstJ)


def _sc_degrees(x, dstJ, ones_hb):
    N, D = x.shape
    nb = dstJ.shape[1]
    rows_per = N // NS
    mesh = plsc.VectorSubcoreMesh(core_axis_name="c", subcore_axis_name="s")

    scratch = [pltpu.VMEM((nb, B), jnp.int32),
               pltpu.VMEM((B, D), jnp.float32),
               pltpu.VMEM_SHARED((N, D), jnp.float32)]

    def body(x_h, dstJ_h, ones_h, out_h, dst_v, ones_v, acc_sh):
        c = lax.axis_index("c")
        s = lax.axis_index("s")
        w = c * NS + s
        pltpu.sync_copy(dstJ_h.at[w], dst_v)
        pltpu.sync_copy(ones_h, ones_v)
        r0 = s * rows_per
        pltpu.sync_copy(x_h.at[pl.ds(r0, rows_per)],
                        acc_sh.at[pl.ds(r0, rows_per)])
        plsc.subcore_barrier()

        def step(j, carry):
            pltpu.sync_copy(ones_v, acc_sh.at[dst_v.at[j]], add=True)
            return carry

        lax.fori_loop(0, nb, step, 0)
        plsc.subcore_barrier()
        pltpu.sync_copy(acc_sh.at[pl.ds(r0, rows_per)],
                        out_h.at[c, pl.ds(r0, rows_per)])

    fn = pl.kernel(body,
                   out_type=jax.ShapeDtypeStruct((2, N, D), jnp.float32),
                   mesh=mesh, scratch_types=scratch)
    return fn(x, dstJ, ones_hb)


def kernel(x, edge_index, W1, b1, W2, b2):
    N, D = x.shape
    E = edge_index.shape[1]

    chunk = -(-(-(-N // NS)) // 8) * 8
    Np = chunk * NS
    if Np != N:
        x = jnp.concatenate([x, jnp.zeros((Np - N, D), x.dtype)])

    nb = math.ceil(E / (NW * B))
    ep = NW * B * nb
    src = edge_index[0]
    dst = edge_index[1]
    if ep != E:
        src = jnp.concatenate([src, jnp.zeros((ep - E,), jnp.int32)])
        dst = jnp.concatenate([dst, jnp.full((ep - E,), N, jnp.int32)])
    srcJ = src.reshape(NW, nb, B)
    dstJ = dst.reshape(NW, nb, B)

    b1_2d = b1.reshape(1, D)
    b2_2d = b2.reshape(1, D)
    rb = Np // 4 if Np % 4 == 0 else Np

    ones_hb = jnp.ones((B, D), jnp.float32)
    dgp = _sc_degrees(x, dstJ, ones_hb)
    g1 = _matmul(x, W1, rb)
    s1p = _sc_messages(g1, srcJ, dstJ)
    g2 = _tc_mid(s1p, g1, dgp, x, b1_2d, W2, rb)
    s2p = _sc_messages(g2, srcJ, dstJ)
    return _tc_final(s2p, g2, dgp, x, b2_2d, rb)[:N]


# speedup vs baseline: 1.2139x; 1.0014x over previous
import math

import jax
import jax.numpy as jnp
from jax import lax
from jax.experimental import pallas as pl
from jax.experimental.pallas import tpu as pltpu
from jax.experimental.pallas import tpu_sc as plsc

NC = 2
NS = 16
NW = NC * NS
B = 128


def _mm_body(x_ref, w_ref, o_ref):
    o_ref[...] = lax.dot_general(
        x_ref[...], w_ref[...], (((1,), (0,)), ((), ())),
        precision=lax.Precision.HIGHEST, preferred_element_type=jnp.float32)


def _matmul(x, W, row_block):
    N, D = x.shape
    return pl.pallas_call(
        _mm_body,
        grid=(N // row_block,),
        in_specs=[pl.BlockSpec((row_block, D), lambda i: (i, 0)),
                  pl.BlockSpec((D, D), lambda i: (0, 0))],
        out_specs=pl.BlockSpec((row_block, D), lambda i: (i, 0)),
        out_shape=jax.ShapeDtypeStruct((N, D), jnp.float32),
    )(x, W)


def _mid_body(sp_ref, g_ref, dgp_ref, x_ref, b_ref, w_ref, o_ref):
    s = sp_ref[0] + sp_ref[1] - g_ref[...]
    deg = (dgp_ref[0, :, 0:1] + dgp_ref[1, :, 0:1]
           - 2.0 * x_ref[:, 0:1] + 1.0)
    h = jnp.maximum(s / deg + b_ref[...], 0.0)
    o_ref[...] = lax.dot_general(
        h, w_ref[...], (((1,), (0,)), ((), ())),
        precision=lax.Precision.HIGHEST, preferred_element_type=jnp.float32)


def _tc_mid(sp, g, dgp, x, b2d, W, row_block):
    N, D = g.shape
    return pl.pallas_call(
        _mid_body,
        grid=(N // row_block,),
        in_specs=[pl.BlockSpec((2, row_block, D), lambda i: (0, i, 0)),
                  pl.BlockSpec((row_block, D), lambda i: (i, 0)),
                  pl.BlockSpec((2, row_block, D), lambda i: (0, i, 0)),
                  pl.BlockSpec((row_block, D), lambda i: (i, 0)),
                  pl.BlockSpec((1, D), lambda i: (0, 0)),
                  pl.BlockSpec((D, D), lambda i: (0, 0))],
        out_specs=pl.BlockSpec((row_block, D), lambda i: (i, 0)),
        out_shape=jax.ShapeDtypeStruct((N, D), jnp.float32),
    )(sp, g, dgp, x, b2d, W)


def _fin_body(sp_ref, g_ref, dgp_ref, x_ref, b_ref, o_ref):
    s = sp_ref[0] + sp_ref[1] - g_ref[...]
    deg = (dgp_ref[0, :, 0:1] + dgp_ref[1, :, 0:1]
           - 2.0 * x_ref[:, 0:1] + 1.0)
    o_ref[...] = s / deg + b_ref[...]


def _tc_final(sp, g, dgp, x, b2d, row_block):
    N, D = g.shape
    return pl.pallas_call(
        _fin_body,
        grid=(N // row_block,),
        in_specs=[pl.BlockSpec((2, row_block, D), lambda i: (0, i, 0)),
                  pl.BlockSpec((row_block, D), lambda i: (i, 0)),
                  pl.BlockSpec((2, row_block, D), lambda i: (0, i, 0)),
                  pl.BlockSpec((row_block, D), lambda i: (i, 0)),
                  pl.BlockSpec((1, D), lambda i: (0, 0))],
        out_specs=pl.BlockSpec((row_block, D), lambda i: (i, 0)),
        out_shape=jax.ShapeDtypeStruct((N, D), jnp.float32),
    )(sp, g, dgp, x, b2d)


def _sc_messages(g, srcJ, dstJ):
    N, D = g.shape
    nb = srcJ.shape[1]
    rows_per = N // NS
    mesh = plsc.VectorSubcoreMesh(core_axis_name="c", subcore_axis_name="s")

    scratch = [pltpu.VMEM((nb, B), jnp.int32),
               pltpu.VMEM((nb, B), jnp.int32),
               pltpu.VMEM((B, D), jnp.float32),
               pltpu.SemaphoreType.DMA,
               pltpu.SemaphoreType.DMA,
               pltpu.VMEM_SHARED((N, D), jnp.float32)]

    def body(g_h, srcJ_h, dstJ_h, out_h, src_v, dst_v, rows_v, sem0, sem1,
             acc_sh):
        c = lax.axis_index("c")
        s = lax.axis_index("s")
        w = c * NS + s
        pltpu.sync_copy(srcJ_h.at[w], src_v)
        pltpu.sync_copy(dstJ_h.at[w], dst_v)
        r0 = s * rows_per
        pltpu.sync_copy(g_h.at[pl.ds(r0, rows_per)],
                        acc_sh.at[pl.ds(r0, rows_per)])
        plsc.subcore_barrier()

        H = B // 2

        def step(j, carry):
            h0 = pltpu.async_copy(g_h.at[src_v.at[j, pl.ds(0, H)]],
                                  rows_v.at[pl.ds(0, H)], sem0)
            h1 = pltpu.async_copy(g_h.at[src_v.at[j, pl.ds(H, H)]],
                                  rows_v.at[pl.ds(H, H)], sem1)
            h0.wait()
            h1.wait()
            pltpu.sync_copy(rows_v, acc_sh.at[dst_v.at[j]], add=True)
            return carry

        lax.fori_loop(0, nb, step, 0)
        plsc.subcore_barrier()
        pltpu.sync_copy(acc_sh.at[pl.ds(r0, rows_per)],
                        out_h.at[c, pl.ds(r0, rows_per)])

    fn = pl.kernel(body,
                   out_type=jax.ShapeDtypeStruct((2, N, D), jnp.float32),
                   mesh=mesh, scratch_types=scratch)
    return fn(g, srcJ, dstJ)


def _sc_degrees(x, dstJ, ones_hb):
    N, D = x.shape
    nb = dstJ.shape[1]
    rows_per = N // NS
    mesh = plsc.VectorSubcoreMesh(core_axis_name="c", subcore_axis_name="s")

    scratch = [pltpu.VMEM((nb, B), jnp.int32),
               pltpu.VMEM((B, D), jnp.float32),
               pltpu.VMEM_SHARED((N, D), jnp.float32)]

    def body(x_h, dstJ_h, ones_h, out_h, dst_v, ones_v, acc_sh):
        c = lax.axis_index("c")
        s = lax.axis_index("s")
        w = c * NS + s
        pltpu.sync_copy(dstJ_h.at[w], dst_v)
        pltpu.sync_copy(ones_h, ones_v)
        r0 = s * rows_per
        pltpu.sync_copy(x_h.at[pl.ds(r0, rows_per)],
                        acc_sh.at[pl.ds(r0, rows_per)])
        plsc.subcore_barrier()

        def step(j, carry):
            pltpu.sync_copy(ones_v, acc_sh.at[dst_v.at[j]], add=True)
            return carry

        lax.fori_loop(0, nb, step, 0)
        plsc.subcore_barrier()
        pltpu.sync_copy(acc_sh.at[pl.ds(r0, rows_per)],
                        out_h.at[c, pl.ds(r0, rows_per)])

    fn = pl.kernel(body,
                   out_type=jax.ShapeDtypeStruct((2, N, D), jnp.float32),
                   mesh=mesh, scratch_types=scratch)
    return fn(x, dstJ, ones_hb)


def kernel(x, edge_index, W1, b1, W2, b2):
    N, D = x.shape
    E = edge_index.shape[1]

    chunk = -(-(-(-N // NS)) // 8) * 8
    Np = chunk * NS
    if Np != N:
        x = jnp.concatenate([x, jnp.zeros((Np - N, D), x.dtype)])

    nb = math.ceil(E / (NW * B))
    ep = NW * B * nb
    src = edge_index[0]
    dst = edge_index[1]
    if ep != E:
        src = jnp.concatenate([src, jnp.zeros((ep - E,), jnp.int32)])
        dst = jnp.concatenate([dst, jnp.full((ep - E,), N, jnp.int32)])
    srcJ = src.reshape(NW, nb, B)
    dstJ = dst.reshape(NW, nb, B)

    b1_2d = b1.reshape(1, D)
    b2_2d = b2.reshape(1, D)
    rb = Np // 4 if Np % 4 == 0 else Np

    ones_hb = jnp.ones((B, D), jnp.float32)
    dgp = _sc_degrees(x, dstJ, ones_hb)
    g1 = _matmul(x, W1, rb)
    s1p = _sc_messages(g1, srcJ, dstJ)
    g2 = _tc_mid(s1p, g1, dgp, x, b1_2d, W2, rb)
    s2p = _sc_messages(g2, srcJ, dstJ)
    return _tc_final(s2p, g2, dgp, x, b2_2d, rb)[:N]
